# Initial kernel scaffold; baseline (speedup 1.0000x reference)
import jax
import jax.numpy as jnp
from jax.experimental import pallas as pl


def kernel(x, edge_index, params):
    raise NotImplementedError("write your pallas kernel here")


# speedup vs baseline: 6.4994x; 6.4994x over previous
import functools

import jax
import jax.numpy as jnp
from jax import lax
from jax.experimental import pallas as pl
from jax.experimental.pallas import tpu as pltpu
from jax.experimental.pallas import tpu_sc as plsc

_N = 10000
_E = 320000
_F = 128
_H = 128
_NUM_LAYERS = 3
_BN_EPS = 1e-5

_NC = 2
_NS = 16
_NW = _NC * _NS
_EPW = _E // _NW
_CH = 80
_NCHUNK = _EPW // _CH
_NPAD = 10240
_ROWS_PT = _NPAD // _NS


def _sc_scatter_body(src_hbm, dst_hbm, h_hbm, zero_hbm, out_hbm,
                     src_v, dst_v, rows_v, agg_sh, gsem):
  c = lax.axis_index("c")
  s = lax.axis_index("s")
  wid = c * _NS + s

  pltpu.sync_copy(zero_hbm.at[pl.ds(s * _ROWS_PT, _ROWS_PT)],
                  agg_sh.at[pl.ds(s * _ROWS_PT, _ROWS_PT)])
  pltpu.sync_copy(src_hbm.at[wid], src_v)
  pltpu.sync_copy(dst_hbm.at[wid], dst_v)
  plsc.subcore_barrier()

  @pl.loop(0, _NCHUNK)
  def _chunk(j):
    pltpu.async_copy(h_hbm.at[src_v.at[j]], rows_v, gsem).wait()
    pltpu.sync_copy(rows_v, agg_sh.at[dst_v.at[j]], add=True)

  plsc.subcore_barrier()
  pltpu.sync_copy(agg_sh.at[pl.ds(s * _ROWS_PT, _ROWS_PT)],
                  out_hbm.at[c, pl.ds(s * _ROWS_PT, _ROWS_PT)])


@jax.jit
def _sc_scatter(src3d, dst3d, h, zero):
  mesh = plsc.VectorSubcoreMesh(core_axis_name="c", subcore_axis_name="s",
                                num_cores=_NC, num_subcores=_NS)
  return pl.kernel(
      _sc_scatter_body,
      out_type=jax.ShapeDtypeStruct((_NC, _NPAD, _F), jnp.float32),
      mesh=mesh,
      scratch_types=[
          pltpu.VMEM((_NCHUNK, _CH), jnp.int32),
          pltpu.VMEM((_NCHUNK, _CH), jnp.int32),
          pltpu.VMEM((_CH, _F), jnp.float32),
          pltpu.VMEM_SHARED((_NPAD, _F), jnp.float32),
          pltpu.SemaphoreType.DMA,
      ],
  )(src3d, dst3d, h, zero)


def _mlp_body(h_ref, agg_ref, wa_ref, wb_ref, ca_ref, da_ref,
              cb_ref, db_ref, co_ref, do_ref, out_ref, *, mid_relu):
  m = h_ref[...] + agg_ref[0] + agg_ref[1]
  t = jnp.dot(m, wa_ref[...], preferred_element_type=jnp.float32)
  t = jnp.maximum(t * ca_ref[...] + da_ref[...], 0.0)
  t = jnp.dot(t, wb_ref[...], preferred_element_type=jnp.float32)
  t = t * cb_ref[...] + db_ref[...]
  if mid_relu:
    t = jnp.maximum(t, 0.0)
  t = t * co_ref[...] + do_ref[...]
  if mid_relu:
    t = jnp.maximum(t, 0.0)
  out_ref[...] = t


def _mlp(h, agg, wa, wb, ca, da, cb, db, co, do_, mid_relu):
  blk = 1000
  grid = (_N // blk,)
  vec = lambda i: (0, 0)
  return pl.pallas_call(
      functools.partial(_mlp_body, mid_relu=mid_relu),
      grid=grid,
      in_specs=[
          pl.BlockSpec((blk, _F), lambda i: (i, 0)),
          pl.BlockSpec((_NC, blk, _F), lambda i: (0, i, 0)),
          pl.BlockSpec((_F, _H), vec),
          pl.BlockSpec((_H, _H), vec),
          pl.BlockSpec((1, _H), vec),
          pl.BlockSpec((1, _H), vec),
          pl.BlockSpec((1, _H), vec),
          pl.BlockSpec((1, _H), vec),
          pl.BlockSpec((1, _H), vec),
          pl.BlockSpec((1, _H), vec),
      ],
      out_specs=pl.BlockSpec((blk, _H), lambda i: (i, 0)),
      out_shape=jax.ShapeDtypeStruct((_N, _H), jnp.float32),
  )(h, agg, wa, wb, ca, da, cb, db, co, do_)


def kernel(x, edge_index, params):
  src3d = edge_index[0].reshape(_NW, _NCHUNK, _CH)
  dst3d = edge_index[1].reshape(_NW, _NCHUNK, _CH)
  zero = jnp.zeros((_NPAD, _F), jnp.float32)
  bn_s = 1.0 / jnp.sqrt(1.0 + _BN_EPS)

  h = x
  for l in range(_NUM_LAYERS):
    agg = _sc_scatter(src3d, dst3d, h, zero)
    sa = params["g%da" % l] * bn_s
    sb = params["g%db" % l] * bn_s
    so = params["g%do" % l] * bn_s
    ca = sa
    da = params["b%da" % l] * sa + params["be%da" % l]
    cb = sb
    db = params["b%db" % l] * sb + params["be%db" % l]
    co = so
    do_ = params["be%do" % l]
    r = lambda v: v.reshape(1, _H)
    h = _mlp(h, agg, params["w%da" % l], params["w%db" % l],
             r(ca), r(da), r(cb), r(db), r(co), r(do_),
             mid_relu=(l < _NUM_LAYERS - 1))
  return h
